# baseline (speedup 1.0000x reference)
import functools

import jax
import jax.numpy as jnp
from jax import lax
from jax.experimental import pallas as pl
from jax.experimental.pallas import tpu as pltpu
from jax.experimental.pallas import tpu_sc as plsc

_INFO = plsc.get_sparse_core_info()
_NC = _INFO.num_cores
_NS = _INFO.num_subcores
_NW = _NC * _NS
_CHUNK = 128


def _emb_call(n_chunks, dim, chunks_per_w):
    mesh = plsc.VectorSubcoreMesh(core_axis_name="c", subcore_axis_name="s")

    @functools.partial(
        pl.kernel,
        mesh=mesh,
        compiler_params=pltpu.CompilerParams(use_tc_tiling_on_sc=False),
        out_type=jax.ShapeDtypeStruct((n_chunks * _CHUNK, dim), jnp.float32),
        scratch_types=[
            pltpu.VMEM((chunks_per_w, _CHUNK), jnp.int32),
            pltpu.VMEM((_CHUNK, dim), jnp.float32),
            pltpu.VMEM((_CHUNK, dim), jnp.float32),
            pltpu.SemaphoreType.DMA,
            pltpu.SemaphoreType.DMA,
            pltpu.SemaphoreType.DMA,
            pltpu.SemaphoreType.DMA,
        ],
    )
    def emb(idx_hbm, w_hbm, out_hbm, idx_v, rows0, rows1, sg0, sg1, sw0, sw1):
        wid = lax.axis_index("s") * _NC + lax.axis_index("c")
        base = wid * chunks_per_w
        pltpu.sync_copy(idx_hbm.at[pl.ds(base, chunks_per_w)], idx_v)

        rows = (rows0, rows1)
        sg = (sg0, sg1)
        sw = (sw0, sw1)

        def gather(j, b):
            pltpu.async_copy(w_hbm.at[idx_v.at[j]], rows[b], sg[b])

        def gather_wait(j, b):
            pltpu.make_async_copy(w_hbm.at[idx_v.at[j]], rows[b], sg[b]).wait()

        def wb(j, b):
            pltpu.async_copy(
                rows[b], out_hbm.at[pl.ds((base + j) * _CHUNK, _CHUNK)], sw[b]
            )

        def wb_wait(j, b):
            pltpu.make_async_copy(
                rows[b], out_hbm.at[pl.ds((base + j) * _CHUNK, _CHUNK)], sw[b]
            ).wait()

        gather(0, 0)

        @pl.loop(0, chunks_per_w, step=2)
        def _(j0):
            for b in range(2):
                j = j0 + b
                gather_wait(j, b)

                @pl.when(j >= 1)
                def _():
                    wb_wait(j - 1, 1 - b)

                @pl.when(j + 1 < chunks_per_w)
                def _():
                    gather(j + 1, 1 - b)

                wb(j, b)

        wb_wait(chunks_per_w - 1, (chunks_per_w - 1) % 2)

    return emb


def kernel(x, weight):
    batch, hist = x.shape
    vocab, dim = weight.shape
    n = batch * hist
    assert n % (_NW * _CHUNK) == 0
    n_chunks = n // _CHUNK
    chunks_per_w = n_chunks // _NW
    assert chunks_per_w % 2 == 0

    idx = x.reshape(n_chunks, _CHUNK).astype(jnp.int32)
    out = _emb_call(n_chunks, dim, chunks_per_w)(idx, weight)
    return out.reshape(batch, hist, dim)


# speedup vs baseline: 1.0445x; 1.0445x over previous
import functools

import jax
import jax.numpy as jnp
from jax import lax
from jax.experimental import pallas as pl
from jax.experimental.pallas import tpu as pltpu
from jax.experimental.pallas import tpu_sc as plsc

_INFO = plsc.get_sparse_core_info()
_NC = _INFO.num_cores
_NS = _INFO.num_subcores
_NW = _NC * _NS
_CHUNK = 128
_GPC = 4


def _emb_call(n_chunks, dim, chunks_per_w):
    mesh = plsc.VectorSubcoreMesh(core_axis_name="c", subcore_axis_name="s")
    n_groups = chunks_per_w // _GPC
    grows = _GPC * _CHUNK

    @functools.partial(
        pl.kernel,
        mesh=mesh,
        compiler_params=pltpu.CompilerParams(use_tc_tiling_on_sc=False),
        out_type=jax.ShapeDtypeStruct((n_chunks * _CHUNK, dim), jnp.float32),
        scratch_types=[
            pltpu.VMEM((chunks_per_w, _CHUNK), jnp.int32),
            pltpu.VMEM((grows, dim), jnp.float32),
            pltpu.VMEM((grows, dim), jnp.float32),
            pltpu.SemaphoreType.DMA,
            pltpu.SemaphoreType.DMA,
            pltpu.SemaphoreType.DMA,
            pltpu.SemaphoreType.DMA,
        ],
    )
    def emb(idx_hbm, w_hbm, out_hbm, idx_v, buf0, buf1, sg0, sg1, sw0, sw1):
        wid = lax.axis_index("s") * _NC + lax.axis_index("c")
        base = wid * chunks_per_w
        pltpu.sync_copy(idx_hbm.at[pl.ds(base, chunks_per_w)], idx_v)

        bufs = (buf0, buf1)
        sg = (sg0, sg1)
        sw = (sw0, sw1)

        def fire_group(g, p):
            for k in range(_GPC):
                pltpu.async_copy(
                    w_hbm.at[idx_v.at[g * _GPC + k]],
                    bufs[p].at[pl.ds(k * _CHUNK, _CHUNK)],
                    sg[p],
                )

        def drain_group(g, p):
            for k in range(_GPC):
                pltpu.make_async_copy(
                    w_hbm.at[idx_v.at[g * _GPC + k]],
                    bufs[p].at[pl.ds(k * _CHUNK, _CHUNK)],
                    sg[p],
                ).wait()

        def wb(g, p):
            pltpu.async_copy(
                bufs[p], out_hbm.at[pl.ds((base + g * _GPC) * _CHUNK, grows)], sw[p]
            )

        def wb_wait(g, p):
            pltpu.make_async_copy(
                bufs[p], out_hbm.at[pl.ds((base + g * _GPC) * _CHUNK, grows)], sw[p]
            ).wait()

        fire_group(0, 0)

        @pl.loop(0, n_groups, step=2)
        def _(g0):
            for p in range(2):
                g = g0 + p

                @pl.when(g >= 1)
                def _():
                    wb_wait(g - 1, 1 - p)

                @pl.when(g + 1 < n_groups)
                def _():
                    fire_group(g + 1, 1 - p)

                drain_group(g, p)
                wb(g, p)

        wb_wait(n_groups - 1, (n_groups - 1) % 2)

    return emb


def kernel(x, weight):
    batch, hist = x.shape
    vocab, dim = weight.shape
    n = batch * hist
    assert n % (_NW * _CHUNK * _GPC) == 0
    n_chunks = n // _CHUNK
    chunks_per_w = n_chunks // _NW

    idx = x.reshape(n_chunks, _CHUNK).astype(jnp.int32)
    out = _emb_call(n_chunks, dim, chunks_per_w)(idx, weight)
    return out.reshape(batch, hist, dim)
